# baseline (speedup 1.0000x reference)
import functools

import jax
import jax.numpy as jnp
from jax import lax
from jax.experimental import pallas as pl
from jax.experimental.pallas import tpu as pltpu
from jax.experimental.pallas import tpu_sc as plsc

_B, _N, _K = 128, 32768, 100
_D = 3
_NC, _NS = 2, 16
_NW = _NC * _NS
_ROWS_W = _B // _NW
_CHUNK = 8192
_NCHUNK = _N // _CHUNK
_VPC = _CHUNK // 16
_POOL = 128
_PV = _POOL // 16
_TRIG = 112
_BUF = _TRIG + 48
_GLEN = 128
_INF = float("inf")


def _iota16():
    return lax.broadcasted_iota(jnp.int32, (16,), 0)


def _minmax_kv(ak, ai, bk, bi):
    m = ak <= bk
    lok = jnp.where(m, ak, bk)
    loi = jnp.where(m, ai, bi)
    hik = jnp.where(m, bk, ak)
    hii = jnp.where(m, bi, ai)
    return lok, loi, hik, hii


def _bitonic_clean(ks, vs):
    n = len(ks)
    if n == 1:
        k, v = plsc.sort_key_val(ks[0], vs[0])
        return [k], [v]
    h = n // 2
    lo_k, lo_v, hi_k, hi_v = [], [], [], []
    for t in range(h):
        lk, li, hk, hi_ = _minmax_kv(ks[t], vs[t], ks[t + h], vs[t + h])
        lo_k.append(lk)
        lo_v.append(li)
        hi_k.append(hk)
        hi_v.append(hi_)
    lo_k, lo_v = _bitonic_clean(lo_k, lo_v)
    hi_k, hi_v = _bitonic_clean(hi_k, hi_v)
    return lo_k + hi_k, lo_v + hi_v


def _merge_sorted(ak, av, bk, bv, keep_low_only):
    n = len(ak)
    rk = [lax.rev(x, (0,)) for x in reversed(bk)]
    rv = [lax.rev(x, (0,)) for x in reversed(bv)]
    lo_k, lo_v, hi_k, hi_v = [], [], [], []
    for t in range(n):
        lk, li, hk, hi_ = _minmax_kv(ak[t], av[t], rk[t], rv[t])
        lo_k.append(lk)
        lo_v.append(li)
        hi_k.append(hk)
        hi_v.append(hi_)
    lo_k, lo_v = _bitonic_clean(lo_k, lo_v)
    if keep_low_only:
        return lo_k, lo_v
    hi_k, hi_v = _bitonic_clean(hi_k, hi_v)
    return lo_k + hi_k, lo_v + hi_v


def _sort_vregs(ks, vs):
    n = len(ks)
    if n == 1:
        k, v = plsc.sort_key_val(ks[0], vs[0])
        return [k], [v]
    h = n // 2
    ak, av = _sort_vregs(ks[:h], vs[:h])
    bk, bv = _sort_vregs(ks[h:], vs[h:])
    return _merge_sorted(ak, av, bk, bv, keep_low_only=False)


def _sc_body(x0_hbm, x1_hbm, x2_hbm, maskf_hbm, pref_hbm, out_hbm,
             c0a, c0b, c0c, c1a, c1b, c1c, pvec_ref,
             pool_val, pool_idx, buf_val, buf_idx,
             shifted, ord_idx, gidx, g0, g1, g2, g3, stage,
             sem0, sem1, gsem):
    cid = lax.axis_index("c")
    sid = lax.axis_index("s")
    wid = sid * _NC + cid
    iota = _iota16()
    zeros16 = jnp.zeros((16,), jnp.int32)
    infv = jnp.full((16,), _INF, jnp.float32)
    pltpu.sync_copy(pref_hbm, pvec_ref)
    pv16 = pvec_ref[pl.ds(0, 16)]
    p0 = pv16[0]
    p1 = pv16[1]
    p2 = pv16[2]

    def start_chunk(buf3, off, sem):
        a, bb, c = buf3
        pltpu.make_async_copy(x0_hbm.at[pl.ds(off, _CHUNK)], a, sem).start()
        pltpu.make_async_copy(x1_hbm.at[pl.ds(off, _CHUNK)], bb, sem).start()
        pltpu.make_async_copy(x2_hbm.at[pl.ds(off, _CHUNK)], c, sem).start()

    def wait_chunk(buf3, off, sem):
        a, bb, c = buf3
        pltpu.make_async_copy(x0_hbm.at[pl.ds(off, _CHUNK)], a, sem).wait()
        pltpu.make_async_copy(x1_hbm.at[pl.ds(off, _CHUNK)], bb, sem).wait()
        pltpu.make_async_copy(x2_hbm.at[pl.ds(off, _CHUNK)], c, sem).wait()

    def clear_buf():
        for t in range(_BUF // 16):
            buf_val[pl.ds(16 * t, 16)] = infv
            buf_idx[pl.ds(16 * t, 16)] = zeros16

    def load_pool():
        pk = [pool_val[pl.ds(16 * t, 16)] for t in range(_PV)]
        pv = [pool_idx[pl.ds(16 * t, 16)] for t in range(_PV)]
        return pk, pv

    def store_pool(pk, pv):
        for t in range(_PV):
            pool_val[pl.ds(16 * t, 16)] = pk[t]
            pool_idx[pl.ds(16 * t, 16)] = pv[t]

    def merge(pos, tau):
        del pos, tau
        bk = [buf_val[pl.ds(16 * t, 16)] for t in range(_PV)]
        bv = [buf_idx[pl.ds(16 * t, 16)] for t in range(_PV)]
        bk, bv = _sort_vregs(bk, bv)
        pk, pv = load_pool()
        pk, pv = _merge_sorted(pk, pv, bk, bv, keep_low_only=True)
        store_pool(pk, pv)
        clear_buf()
        new_tau = jnp.max(pk[_PV - 1])
        return jnp.int32(0), new_tau

    def row_body(r, _):
        b = wid * _ROWS_W + r
        for t in range(_PV):
            pool_val[pl.ds(16 * t, 16)] = infv
            pool_idx[pl.ds(16 * t, 16)] = zeros16
        clear_buf()
        rb = b * _N
        start_chunk((c0a, c0b, c0c), rb, sem0)
        start_chunk((c1a, c1b, c1c), rb + _CHUNK, sem1)

        def vreg_step(buf3, base):
            a, bb, c = buf3

            def step(j, carry):
                pos, tau = carry
                sl = pl.ds(j * 16, 16)
                v = (p0 * a[sl] + p1 * bb[sl]) + p2 * c[sl]
                m = v <= tau

                def append(pos, tau):
                    cnt = plsc.all_reduce_population_count(m)[0]
                    plsc.store_compressed(buf_val.at[pl.ds(pos, 16)], v,
                                          mask=m)
                    idxv = base + j * 16 + iota
                    plsc.store_compressed(buf_idx.at[pl.ds(pos, 16)], idxv,
                                          mask=m)
                    pos = pos + cnt
                    return lax.cond(pos >= _TRIG, merge,
                                    lambda p, t: (p, t), pos, tau)

                return lax.cond(jnp.any(m), append,
                                lambda p, t: (p, t), pos, tau)
            return step

        pos = jnp.int32(0)
        tau = jnp.float32(_INF)
        for c in range(_NCHUNK):
            buf3 = (c0a, c0b, c0c) if c % 2 == 0 else (c1a, c1b, c1c)
            csem = sem0 if c % 2 == 0 else sem1
            wait_chunk(buf3, rb + c * _CHUNK, csem)
            pos, tau = lax.fori_loop(
                0, _VPC, vreg_step(buf3, jnp.int32(c * _CHUNK)), (pos, tau),
                unroll=4)
            if c + 2 < _NCHUNK:
                start_chunk(buf3, rb + (c + 2) * _CHUNK, csem)
        pos, tau = merge(pos, tau)

        pk, pv = load_pool()
        neg = jnp.full((16,), jnp.float32(-1.0), jnp.float32)
        plsc.store_scatter(shifted, [iota], neg, mask=(iota < 1))
        for t in range(_PV):
            plsc.store_scatter(shifted, [16 * t + 1 + iota], pk[t])
        ties = jnp.int32(0)
        for t in range(_PV):
            sh = shifted[pl.ds(16 * t, 16)]
            ties = ties + plsc.all_reduce_population_count(pk[t] == sh)[0]

        def fast_order(_):
            for t in range(_PV):
                ord_idx[pl.ds(16 * t, 16)] = pv[t]
            return jnp.int32(0)

        def slow_order(_):
            def rank_one(i, acc):
                base = (i // 16) * 16
                lane = i - base
                hit = iota == lane
                vv = pool_val[pl.ds(base, 16)]
                iv = pool_idx[pl.ds(base, 16)]
                vi = jnp.max(jnp.where(hit, vv, -jnp.inf))
                ii = jnp.max(jnp.where(hit, iv, -2147483647))
                rank = jnp.int32(0)
                for t in range(_PV):
                    less = (pk[t] < vi) | ((pk[t] == vi) & (pv[t] < ii))
                    rank = rank + plsc.all_reduce_population_count(less)[0]
                rs = jnp.full((16,), rank, jnp.int32)
                plsc.store_scatter(ord_idx, [rs],
                                   jnp.full((16,), ii, jnp.int32),
                                   mask=(iota < 1))
                return acc
            return lax.fori_loop(0, _POOL, rank_one, jnp.int32(0))

        lax.cond(ties > 0, slow_order, fast_order, jnp.int32(0))

        for t in range(_PV):
            n = ord_idx[pl.ds(16 * t, 16)]
            gidx[pl.ds(16 * t, 16)] = rb + n
        pltpu.async_copy(x0_hbm.at[gidx], g0, gsem).wait()
        pltpu.async_copy(x1_hbm.at[gidx], g1, gsem).wait()
        pltpu.async_copy(x2_hbm.at[gidx], g2, gsem).wait()
        pltpu.async_copy(maskf_hbm.at[gidx], g3, gsem).wait()

        for t in range(7):
            p = 16 * t + iota
            sl = pl.ds(16 * t, 16)
            plsc.store_scatter(stage, [4 * p], g0[sl], mask=(p < _K))
            plsc.store_scatter(stage, [4 * p + 1], g1[sl], mask=(p < _K))
            plsc.store_scatter(stage, [4 * p + 2], g2[sl], mask=(p < _K))
            plsc.store_scatter(stage, [4 * p + 3], g3[sl], mask=(p < _K))
        pltpu.sync_copy(stage.at[pl.ds(0, 4 * _K)],
                        out_hbm.at[pl.ds(b * 4 * _K, 4 * _K)])
        return 0

    lax.fori_loop(0, _ROWS_W, row_body, 0)


def _build_sc():
    return pl.kernel(
        _sc_body,
        out_type=jax.ShapeDtypeStruct((_B * 4 * _K,), jnp.float32),
        mesh=plsc.VectorSubcoreMesh(core_axis_name="c", subcore_axis_name="s",
                                    num_cores=_NC, num_subcores=_NS),
        compiler_params=pltpu.CompilerParams(needs_layout_passes=False),
        scratch_types=[
            pltpu.VMEM((_CHUNK,), jnp.float32),
            pltpu.VMEM((_CHUNK,), jnp.float32),
            pltpu.VMEM((_CHUNK,), jnp.float32),
            pltpu.VMEM((_CHUNK,), jnp.float32),
            pltpu.VMEM((_CHUNK,), jnp.float32),
            pltpu.VMEM((_CHUNK,), jnp.float32),
            pltpu.VMEM((16,), jnp.float32),
            pltpu.VMEM((_POOL,), jnp.float32),
            pltpu.VMEM((_POOL,), jnp.int32),
            pltpu.VMEM((_BUF,), jnp.float32),
            pltpu.VMEM((_BUF,), jnp.int32),
            pltpu.VMEM((_POOL + 16,), jnp.float32),
            pltpu.VMEM((_POOL,), jnp.int32),
            pltpu.VMEM((_GLEN,), jnp.int32),
            pltpu.VMEM((_GLEN,), jnp.float32),
            pltpu.VMEM((_GLEN,), jnp.float32),
            pltpu.VMEM((_GLEN,), jnp.float32),
            pltpu.VMEM((_GLEN,), jnp.float32),
            pltpu.VMEM((512,), jnp.float32),
            pltpu.SemaphoreType.DMA,
            pltpu.SemaphoreType.DMA,
            pltpu.SemaphoreType.DMA,
        ],
    )


@jax.jit
def _run(sols, sols_mask, pref):
    x0 = sols[:, :, 0].reshape(_B * _N)
    x1 = sols[:, :, 1].reshape(_B * _N)
    x2 = sols[:, :, 2].reshape(_B * _N)
    maskf = sols_mask.reshape(_B * _N)
    pref16 = jnp.pad(pref, (0, 16 - _D))
    out = _build_sc()(x0, x1, x2, maskf, pref16)
    return out.reshape(_B, _K, 4)


def kernel(sols, sols_mask, pref, k):
    del k
    return _run(sols, sols_mask, pref)


# speedup vs baseline: 1.4235x; 1.4235x over previous
import functools

import jax
import jax.numpy as jnp
from jax import lax
from jax.experimental import pallas as pl
from jax.experimental.pallas import tpu as pltpu
from jax.experimental.pallas import tpu_sc as plsc

_B, _N, _K = 128, 32768, 100
_D = 3
_NC, _NS = 2, 16
_NW = _NC * _NS
_ROWS_W = _B // _NW
_CHUNK = 8192
_NCHUNK = _N // _CHUNK
_VPC = _CHUNK // 16
_POOL = 128
_PV = _POOL // 16
_TRIG = 112
_BUF = _TRIG + 48
_GLEN = 128
_INF = float("inf")


def _iota16():
    return lax.broadcasted_iota(jnp.int32, (16,), 0)


def _minmax_kv(ak, ai, bk, bi):
    m = ak <= bk
    lok = jnp.where(m, ak, bk)
    loi = jnp.where(m, ai, bi)
    hik = jnp.where(m, bk, ak)
    hii = jnp.where(m, bi, ai)
    return lok, loi, hik, hii


def _bitonic_clean(ks, vs):
    n = len(ks)
    if n == 1:
        k, v = plsc.sort_key_val(ks[0], vs[0])
        return [k], [v]
    h = n // 2
    lo_k, lo_v, hi_k, hi_v = [], [], [], []
    for t in range(h):
        lk, li, hk, hi_ = _minmax_kv(ks[t], vs[t], ks[t + h], vs[t + h])
        lo_k.append(lk)
        lo_v.append(li)
        hi_k.append(hk)
        hi_v.append(hi_)
    lo_k, lo_v = _bitonic_clean(lo_k, lo_v)
    hi_k, hi_v = _bitonic_clean(hi_k, hi_v)
    return lo_k + hi_k, lo_v + hi_v


def _merge_sorted(ak, av, bk, bv, keep_low_only):
    n = len(ak)
    rk = [lax.rev(x, (0,)) for x in reversed(bk)]
    rv = [lax.rev(x, (0,)) for x in reversed(bv)]
    lo_k, lo_v, hi_k, hi_v = [], [], [], []
    for t in range(n):
        lk, li, hk, hi_ = _minmax_kv(ak[t], av[t], rk[t], rv[t])
        lo_k.append(lk)
        lo_v.append(li)
        hi_k.append(hk)
        hi_v.append(hi_)
    lo_k, lo_v = _bitonic_clean(lo_k, lo_v)
    if keep_low_only:
        return lo_k, lo_v
    hi_k, hi_v = _bitonic_clean(hi_k, hi_v)
    return lo_k + hi_k, lo_v + hi_v


def _sort_vregs(ks, vs):
    n = len(ks)
    if n == 1:
        k, v = plsc.sort_key_val(ks[0], vs[0])
        return [k], [v]
    h = n // 2
    ak, av = _sort_vregs(ks[:h], vs[:h])
    bk, bv = _sort_vregs(ks[h:], vs[h:])
    return _merge_sorted(ak, av, bk, bv, keep_low_only=False)


def _sc_body(x0_hbm, x1_hbm, x2_hbm, maskf_hbm, pref_hbm, out_hbm,
             c0a, c0b, c0c, c1a, c1b, c1c, pvec_ref,
             pool_val, pool_idx, buf_val, buf_idx,
             shifted, ord_idx, gidx, g0, g1, g2, g3, stage,
             sem0, sem1, gsem):
    cid = lax.axis_index("c")
    sid = lax.axis_index("s")
    wid = sid * _NC + cid
    iota = _iota16()
    zeros16 = jnp.zeros((16,), jnp.int32)
    infv = jnp.full((16,), _INF, jnp.float32)
    pltpu.sync_copy(pref_hbm, pvec_ref)
    pv16 = pvec_ref[pl.ds(0, 16)]
    p0 = pv16[0]
    p1 = pv16[1]
    p2 = pv16[2]

    def start_chunk(buf3, off, sem):
        a, bb, c = buf3
        pltpu.make_async_copy(x0_hbm.at[pl.ds(off, _CHUNK)], a, sem).start()
        pltpu.make_async_copy(x1_hbm.at[pl.ds(off, _CHUNK)], bb, sem).start()
        pltpu.make_async_copy(x2_hbm.at[pl.ds(off, _CHUNK)], c, sem).start()

    def wait_chunk(buf3, off, sem):
        a, bb, c = buf3
        pltpu.make_async_copy(x0_hbm.at[pl.ds(off, _CHUNK)], a, sem).wait()
        pltpu.make_async_copy(x1_hbm.at[pl.ds(off, _CHUNK)], bb, sem).wait()
        pltpu.make_async_copy(x2_hbm.at[pl.ds(off, _CHUNK)], c, sem).wait()

    def clear_buf():
        for t in range(_BUF // 16):
            buf_val[pl.ds(16 * t, 16)] = infv
            buf_idx[pl.ds(16 * t, 16)] = zeros16

    def load_pool():
        pk = [pool_val[pl.ds(16 * t, 16)] for t in range(_PV)]
        pv = [pool_idx[pl.ds(16 * t, 16)] for t in range(_PV)]
        return pk, pv

    def store_pool(pk, pv):
        for t in range(_PV):
            pool_val[pl.ds(16 * t, 16)] = pk[t]
            pool_idx[pl.ds(16 * t, 16)] = pv[t]

    def merge(pos, tau):
        del pos, tau
        bk = [buf_val[pl.ds(16 * t, 16)] for t in range(_PV)]
        bv = [buf_idx[pl.ds(16 * t, 16)] for t in range(_PV)]
        bk, bv = _sort_vregs(bk, bv)
        pk, pv = load_pool()
        pk, pv = _merge_sorted(pk, pv, bk, bv, keep_low_only=True)
        store_pool(pk, pv)
        clear_buf()
        new_tau = jnp.max(pk[_PV - 1])
        return jnp.int32(0), new_tau

    def row_body(r, _):
        b = wid * _ROWS_W + r
        for t in range(_PV):
            pool_val[pl.ds(16 * t, 16)] = infv
            pool_idx[pl.ds(16 * t, 16)] = zeros16
        clear_buf()
        rb = b * _N
        start_chunk((c0a, c0b, c0c), rb, sem0)
        start_chunk((c1a, c1b, c1c), rb + _CHUNK, sem1)

        def vreg_step(buf3, base):
            a, bb, c = buf3

            def step(j, carry):
                pos, tau = carry
                sl = pl.ds(j * 16, 16)
                v = (p0 * a[sl] + p1 * bb[sl]) + p2 * c[sl]
                m = v <= tau
                cnt = plsc.all_reduce_population_count(m)[0]
                plsc.store_compressed(buf_val.at[pl.ds(pos, 16)], v, mask=m)
                idxv = base + j * 16 + iota
                plsc.store_compressed(buf_idx.at[pl.ds(pos, 16)], idxv, mask=m)
                pos = pos + cnt
                return lax.cond(pos >= _TRIG, merge,
                                lambda p, t: (p, t), pos, tau)
            return step

        pos = jnp.int32(0)
        tau = jnp.float32(_INF)
        for c in range(_NCHUNK):
            buf3 = (c0a, c0b, c0c) if c % 2 == 0 else (c1a, c1b, c1c)
            csem = sem0 if c % 2 == 0 else sem1
            wait_chunk(buf3, rb + c * _CHUNK, csem)
            pos, tau = lax.fori_loop(
                0, _VPC, vreg_step(buf3, jnp.int32(c * _CHUNK)), (pos, tau),
                unroll=4)
            if c + 2 < _NCHUNK:
                start_chunk(buf3, rb + (c + 2) * _CHUNK, csem)
        pos, tau = merge(pos, tau)

        pk, pv = load_pool()
        neg = jnp.full((16,), jnp.float32(-1.0), jnp.float32)
        plsc.store_scatter(shifted, [iota], neg, mask=(iota < 1))
        for t in range(_PV):
            plsc.store_scatter(shifted, [16 * t + 1 + iota], pk[t])
        ties = jnp.int32(0)
        for t in range(_PV):
            sh = shifted[pl.ds(16 * t, 16)]
            ties = ties + plsc.all_reduce_population_count(pk[t] == sh)[0]

        def fast_order(_):
            for t in range(_PV):
                ord_idx[pl.ds(16 * t, 16)] = pv[t]
            return jnp.int32(0)

        def slow_order(_):
            def rank_one(i, acc):
                base = (i // 16) * 16
                lane = i - base
                hit = iota == lane
                vv = pool_val[pl.ds(base, 16)]
                iv = pool_idx[pl.ds(base, 16)]
                vi = jnp.max(jnp.where(hit, vv, -jnp.inf))
                ii = jnp.max(jnp.where(hit, iv, -2147483647))
                rank = jnp.int32(0)
                for t in range(_PV):
                    less = (pk[t] < vi) | ((pk[t] == vi) & (pv[t] < ii))
                    rank = rank + plsc.all_reduce_population_count(less)[0]
                rs = jnp.full((16,), rank, jnp.int32)
                plsc.store_scatter(ord_idx, [rs],
                                   jnp.full((16,), ii, jnp.int32),
                                   mask=(iota < 1))
                return acc
            return lax.fori_loop(0, _POOL, rank_one, jnp.int32(0))

        lax.cond(ties > 0, slow_order, fast_order, jnp.int32(0))

        for t in range(_PV):
            n = ord_idx[pl.ds(16 * t, 16)]
            gidx[pl.ds(16 * t, 16)] = rb + n
        pltpu.async_copy(x0_hbm.at[gidx], g0, gsem).wait()
        pltpu.async_copy(x1_hbm.at[gidx], g1, gsem).wait()
        pltpu.async_copy(x2_hbm.at[gidx], g2, gsem).wait()
        pltpu.async_copy(maskf_hbm.at[gidx], g3, gsem).wait()

        for t in range(7):
            p = 16 * t + iota
            sl = pl.ds(16 * t, 16)
            plsc.store_scatter(stage, [4 * p], g0[sl], mask=(p < _K))
            plsc.store_scatter(stage, [4 * p + 1], g1[sl], mask=(p < _K))
            plsc.store_scatter(stage, [4 * p + 2], g2[sl], mask=(p < _K))
            plsc.store_scatter(stage, [4 * p + 3], g3[sl], mask=(p < _K))
        pltpu.sync_copy(stage.at[pl.ds(0, 4 * _K)],
                        out_hbm.at[pl.ds(b * 4 * _K, 4 * _K)])
        return 0

    lax.fori_loop(0, _ROWS_W, row_body, 0)


def _build_sc():
    return pl.kernel(
        _sc_body,
        out_type=jax.ShapeDtypeStruct((_B * 4 * _K,), jnp.float32),
        mesh=plsc.VectorSubcoreMesh(core_axis_name="c", subcore_axis_name="s",
                                    num_cores=_NC, num_subcores=_NS),
        compiler_params=pltpu.CompilerParams(needs_layout_passes=False),
        scratch_types=[
            pltpu.VMEM((_CHUNK,), jnp.float32),
            pltpu.VMEM((_CHUNK,), jnp.float32),
            pltpu.VMEM((_CHUNK,), jnp.float32),
            pltpu.VMEM((_CHUNK,), jnp.float32),
            pltpu.VMEM((_CHUNK,), jnp.float32),
            pltpu.VMEM((_CHUNK,), jnp.float32),
            pltpu.VMEM((16,), jnp.float32),
            pltpu.VMEM((_POOL,), jnp.float32),
            pltpu.VMEM((_POOL,), jnp.int32),
            pltpu.VMEM((_BUF,), jnp.float32),
            pltpu.VMEM((_BUF,), jnp.int32),
            pltpu.VMEM((_POOL + 16,), jnp.float32),
            pltpu.VMEM((_POOL,), jnp.int32),
            pltpu.VMEM((_GLEN,), jnp.int32),
            pltpu.VMEM((_GLEN,), jnp.float32),
            pltpu.VMEM((_GLEN,), jnp.float32),
            pltpu.VMEM((_GLEN,), jnp.float32),
            pltpu.VMEM((_GLEN,), jnp.float32),
            pltpu.VMEM((512,), jnp.float32),
            pltpu.SemaphoreType.DMA,
            pltpu.SemaphoreType.DMA,
            pltpu.SemaphoreType.DMA,
        ],
    )


@jax.jit
def _run(sols, sols_mask, pref):
    x0 = sols[:, :, 0].reshape(_B * _N)
    x1 = sols[:, :, 1].reshape(_B * _N)
    x2 = sols[:, :, 2].reshape(_B * _N)
    maskf = sols_mask.reshape(_B * _N)
    pref16 = jnp.pad(pref, (0, 16 - _D))
    out = _build_sc()(x0, x1, x2, maskf, pref16)
    return out.reshape(_B, _K, 4)


def kernel(sols, sols_mask, pref, k):
    del k
    return _run(sols, sols_mask, pref)


# speedup vs baseline: 1.8873x; 1.3258x over previous
import functools

import jax
import jax.numpy as jnp
from jax import lax
from jax.experimental import pallas as pl
from jax.experimental.pallas import tpu as pltpu
from jax.experimental.pallas import tpu_sc as plsc

_B, _N, _K = 128, 32768, 100
_D = 3
_NC, _NS = 2, 16
_NW = _NC * _NS
_ROWS_W = _B // _NW
_CHUNK = 8192
_NCHUNK = _N // _CHUNK
_VPC = _CHUNK // 16
_POOL = 128
_PV = _POOL // 16
_TRIG = 112
_GRP = 8
_BUF = _TRIG + 16 * _GRP + 16
_GLEN = 128
_INF = float("inf")


def _iota16():
    return lax.broadcasted_iota(jnp.int32, (16,), 0)


def _minmax_kv(ak, ai, bk, bi):
    m = ak <= bk
    lok = jnp.where(m, ak, bk)
    loi = jnp.where(m, ai, bi)
    hik = jnp.where(m, bk, ak)
    hii = jnp.where(m, bi, ai)
    return lok, loi, hik, hii


def _bitonic_clean(ks, vs):
    n = len(ks)
    if n == 1:
        k, v = plsc.sort_key_val(ks[0], vs[0])
        return [k], [v]
    h = n // 2
    lo_k, lo_v, hi_k, hi_v = [], [], [], []
    for t in range(h):
        lk, li, hk, hi_ = _minmax_kv(ks[t], vs[t], ks[t + h], vs[t + h])
        lo_k.append(lk)
        lo_v.append(li)
        hi_k.append(hk)
        hi_v.append(hi_)
    lo_k, lo_v = _bitonic_clean(lo_k, lo_v)
    hi_k, hi_v = _bitonic_clean(hi_k, hi_v)
    return lo_k + hi_k, lo_v + hi_v


def _merge_sorted(ak, av, bk, bv, keep_low_only):
    n = len(ak)
    rk = [lax.rev(x, (0,)) for x in reversed(bk)]
    rv = [lax.rev(x, (0,)) for x in reversed(bv)]
    lo_k, lo_v, hi_k, hi_v = [], [], [], []
    for t in range(n):
        lk, li, hk, hi_ = _minmax_kv(ak[t], av[t], rk[t], rv[t])
        lo_k.append(lk)
        lo_v.append(li)
        hi_k.append(hk)
        hi_v.append(hi_)
    lo_k, lo_v = _bitonic_clean(lo_k, lo_v)
    if keep_low_only:
        return lo_k, lo_v
    hi_k, hi_v = _bitonic_clean(hi_k, hi_v)
    return lo_k + hi_k, lo_v + hi_v


def _sort_vregs(ks, vs):
    n = len(ks)
    if n == 1:
        k, v = plsc.sort_key_val(ks[0], vs[0])
        return [k], [v]
    h = n // 2
    ak, av = _sort_vregs(ks[:h], vs[:h])
    bk, bv = _sort_vregs(ks[h:], vs[h:])
    return _merge_sorted(ak, av, bk, bv, keep_low_only=False)


def _sc_body(x0_hbm, x1_hbm, x2_hbm, maskf_hbm, pref_hbm, out_hbm,
             c0a, c0b, c0c, c1a, c1b, c1c, pvec_ref,
             pool_val, pool_idx, buf_val, buf_idx,
             shifted, ord_idx, gidx, g0, g1, g2, g3, stage,
             sem0, sem1, gsem):
    cid = lax.axis_index("c")
    sid = lax.axis_index("s")
    wid = sid * _NC + cid
    iota = _iota16()
    zeros16 = jnp.zeros((16,), jnp.int32)
    infv = jnp.full((16,), _INF, jnp.float32)
    pltpu.sync_copy(pref_hbm, pvec_ref)
    pv16 = pvec_ref[pl.ds(0, 16)]
    p0 = pv16[0]
    p1 = pv16[1]
    p2 = pv16[2]

    def start_chunk(buf3, off, sem):
        a, bb, c = buf3
        pltpu.make_async_copy(x0_hbm.at[pl.ds(off, _CHUNK)], a, sem).start()
        pltpu.make_async_copy(x1_hbm.at[pl.ds(off, _CHUNK)], bb, sem).start()
        pltpu.make_async_copy(x2_hbm.at[pl.ds(off, _CHUNK)], c, sem).start()

    def wait_chunk(buf3, off, sem):
        a, bb, c = buf3
        pltpu.make_async_copy(x0_hbm.at[pl.ds(off, _CHUNK)], a, sem).wait()
        pltpu.make_async_copy(x1_hbm.at[pl.ds(off, _CHUNK)], bb, sem).wait()
        pltpu.make_async_copy(x2_hbm.at[pl.ds(off, _CHUNK)], c, sem).wait()

    def clear_buf():
        for t in range(_BUF // 16):
            buf_val[pl.ds(16 * t, 16)] = infv
            buf_idx[pl.ds(16 * t, 16)] = zeros16

    def load_pool():
        pk = [pool_val[pl.ds(16 * t, 16)] for t in range(_PV)]
        pv = [pool_idx[pl.ds(16 * t, 16)] for t in range(_PV)]
        return pk, pv

    def store_pool(pk, pv):
        for t in range(_PV):
            pool_val[pl.ds(16 * t, 16)] = pk[t]
            pool_idx[pl.ds(16 * t, 16)] = pv[t]

    def merge(pos, tau):
        del pos, tau
        bk = [buf_val[pl.ds(16 * t, 16)] for t in range(_PV)]
        bv = [buf_idx[pl.ds(16 * t, 16)] for t in range(_PV)]
        bk, bv = _sort_vregs(bk, bv)
        pk, pv = load_pool()
        pk, pv = _merge_sorted(pk, pv, bk, bv, keep_low_only=True)
        store_pool(pk, pv)
        clear_buf()
        new_tau = jnp.max(pk[_PV - 1])
        return jnp.int32(0), new_tau

    def row_body(r, _):
        b = wid * _ROWS_W + r
        for t in range(_PV):
            pool_val[pl.ds(16 * t, 16)] = infv
            pool_idx[pl.ds(16 * t, 16)] = zeros16
        clear_buf()
        rb = b * _N
        start_chunk((c0a, c0b, c0c), rb, sem0)
        start_chunk((c1a, c1b, c1c), rb + _CHUNK, sem1)

        def vreg_step(buf3, base):
            a, bb, c = buf3

            def step(j8, carry):
                pos, tau = carry
                for u in range(_GRP):
                    j = j8 * _GRP + u
                    sl = pl.ds(j * 16, 16)
                    v = (p0 * a[sl] + p1 * bb[sl]) + p2 * c[sl]
                    m = v <= tau
                    cnt = plsc.all_reduce_population_count(m)[0]
                    plsc.store_compressed(buf_val.at[pl.ds(pos, 16)], v,
                                          mask=m)
                    idxv = base + j * 16 + iota
                    plsc.store_compressed(buf_idx.at[pl.ds(pos, 16)], idxv,
                                          mask=m)
                    pos = pos + cnt
                return lax.cond(pos >= _TRIG, merge,
                                lambda p, t: (p, t), pos, tau)
            return step

        pos = jnp.int32(0)
        tau = jnp.float32(_INF)
        for c in range(_NCHUNK):
            buf3 = (c0a, c0b, c0c) if c % 2 == 0 else (c1a, c1b, c1c)
            csem = sem0 if c % 2 == 0 else sem1
            wait_chunk(buf3, rb + c * _CHUNK, csem)
            pos, tau = lax.fori_loop(
                0, _VPC // _GRP, vreg_step(buf3, jnp.int32(c * _CHUNK)),
                (pos, tau))
            if c + 2 < _NCHUNK:
                start_chunk(buf3, rb + (c + 2) * _CHUNK, csem)
        pos, tau = merge(pos, tau)

        pk, pv = load_pool()
        neg = jnp.full((16,), jnp.float32(-1.0), jnp.float32)
        plsc.store_scatter(shifted, [iota], neg, mask=(iota < 1))
        for t in range(_PV):
            plsc.store_scatter(shifted, [16 * t + 1 + iota], pk[t])
        ties = jnp.int32(0)
        for t in range(_PV):
            sh = shifted[pl.ds(16 * t, 16)]
            ties = ties + plsc.all_reduce_population_count(pk[t] == sh)[0]

        def fast_order(_):
            for t in range(_PV):
                ord_idx[pl.ds(16 * t, 16)] = pv[t]
            return jnp.int32(0)

        def slow_order(_):
            def rank_one(i, acc):
                base = (i // 16) * 16
                lane = i - base
                hit = iota == lane
                vv = pool_val[pl.ds(base, 16)]
                iv = pool_idx[pl.ds(base, 16)]
                vi = jnp.max(jnp.where(hit, vv, -jnp.inf))
                ii = jnp.max(jnp.where(hit, iv, -2147483647))
                rank = jnp.int32(0)
                for t in range(_PV):
                    less = (pk[t] < vi) | ((pk[t] == vi) & (pv[t] < ii))
                    rank = rank + plsc.all_reduce_population_count(less)[0]
                rs = jnp.full((16,), rank, jnp.int32)
                plsc.store_scatter(ord_idx, [rs],
                                   jnp.full((16,), ii, jnp.int32),
                                   mask=(iota < 1))
                return acc
            return lax.fori_loop(0, _POOL, rank_one, jnp.int32(0))

        lax.cond(ties > 0, slow_order, fast_order, jnp.int32(0))

        for t in range(_PV):
            n = ord_idx[pl.ds(16 * t, 16)]
            gidx[pl.ds(16 * t, 16)] = rb + n
        pltpu.async_copy(x0_hbm.at[gidx], g0, gsem).wait()
        pltpu.async_copy(x1_hbm.at[gidx], g1, gsem).wait()
        pltpu.async_copy(x2_hbm.at[gidx], g2, gsem).wait()
        pltpu.async_copy(maskf_hbm.at[gidx], g3, gsem).wait()

        for t in range(7):
            p = 16 * t + iota
            sl = pl.ds(16 * t, 16)
            plsc.store_scatter(stage, [4 * p], g0[sl], mask=(p < _K))
            plsc.store_scatter(stage, [4 * p + 1], g1[sl], mask=(p < _K))
            plsc.store_scatter(stage, [4 * p + 2], g2[sl], mask=(p < _K))
            plsc.store_scatter(stage, [4 * p + 3], g3[sl], mask=(p < _K))
        pltpu.sync_copy(stage.at[pl.ds(0, 4 * _K)],
                        out_hbm.at[pl.ds(b * 4 * _K, 4 * _K)])
        return 0

    lax.fori_loop(0, _ROWS_W, row_body, 0)


def _build_sc():
    return pl.kernel(
        _sc_body,
        out_type=jax.ShapeDtypeStruct((_B * 4 * _K,), jnp.float32),
        mesh=plsc.VectorSubcoreMesh(core_axis_name="c", subcore_axis_name="s",
                                    num_cores=_NC, num_subcores=_NS),
        compiler_params=pltpu.CompilerParams(needs_layout_passes=False),
        scratch_types=[
            pltpu.VMEM((_CHUNK,), jnp.float32),
            pltpu.VMEM((_CHUNK,), jnp.float32),
            pltpu.VMEM((_CHUNK,), jnp.float32),
            pltpu.VMEM((_CHUNK,), jnp.float32),
            pltpu.VMEM((_CHUNK,), jnp.float32),
            pltpu.VMEM((_CHUNK,), jnp.float32),
            pltpu.VMEM((16,), jnp.float32),
            pltpu.VMEM((_POOL,), jnp.float32),
            pltpu.VMEM((_POOL,), jnp.int32),
            pltpu.VMEM((_BUF,), jnp.float32),
            pltpu.VMEM((_BUF,), jnp.int32),
            pltpu.VMEM((_POOL + 16,), jnp.float32),
            pltpu.VMEM((_POOL,), jnp.int32),
            pltpu.VMEM((_GLEN,), jnp.int32),
            pltpu.VMEM((_GLEN,), jnp.float32),
            pltpu.VMEM((_GLEN,), jnp.float32),
            pltpu.VMEM((_GLEN,), jnp.float32),
            pltpu.VMEM((_GLEN,), jnp.float32),
            pltpu.VMEM((512,), jnp.float32),
            pltpu.SemaphoreType.DMA,
            pltpu.SemaphoreType.DMA,
            pltpu.SemaphoreType.DMA,
        ],
    )


@jax.jit
def _run(sols, sols_mask, pref):
    x0 = sols[:, :, 0].reshape(_B * _N)
    x1 = sols[:, :, 1].reshape(_B * _N)
    x2 = sols[:, :, 2].reshape(_B * _N)
    maskf = sols_mask.reshape(_B * _N)
    pref16 = jnp.pad(pref, (0, 16 - _D))
    out = _build_sc()(x0, x1, x2, maskf, pref16)
    return out.reshape(_B, _K, 4)


def kernel(sols, sols_mask, pref, k):
    del k
    return _run(sols, sols_mask, pref)
